# Initial kernel scaffold; baseline (speedup 1.0000x reference)
import jax
import jax.numpy as jnp
from jax.experimental import pallas as pl


def kernel(x, edge_attr, edge_index, W_e, b_e, W_n, b_n):
    raise NotImplementedError("write your pallas kernel here")


# speedup vs baseline: 6.6428x; 6.6428x over previous
import functools

import jax
import jax.numpy as jnp
from jax import lax
from jax.experimental import pallas as pl
from jax.experimental.pallas import tpu as pltpu
from jax.experimental.pallas import tpu_sc as plsc

N = 10000
E = 320000
DF = 128
DE = 16
HE = 16
HN = 128

NC = 2
NS = 16
NW = NC * NS
EPW = E // NW
EB = 1000
NPS = 1000


def _proj_body(x_ref, w_ref, xs_ref, xr_ref):
    r = jnp.dot(x_ref[...], w_ref[...], preferred_element_type=jnp.float32)
    xs_ref[...] = r[:, :HE]
    xr_ref[...] = r[:, HE:]


def _node_proj(x, w_sr):
    blk = 1000
    return pl.pallas_call(
        _proj_body,
        grid=(N // blk,),
        in_specs=[
            pl.BlockSpec((blk, DF), lambda i: (i, 0)),
            pl.BlockSpec((DF, 2 * HE), lambda i: (0, 0)),
        ],
        out_specs=[
            pl.BlockSpec((blk, HE), lambda i: (i, 0)),
            pl.BlockSpec((blk, HE), lambda i: (i, 0)),
        ],
        out_shape=[
            jax.ShapeDtypeStruct((N, HE), jnp.float32),
            jax.ShapeDtypeStruct((N, HE), jnp.float32),
        ],
    )(x, w_sr)


def _eproj_body(ea_ref, w_ref, b_ref, o_ref):
    o_ref[...] = (
        jnp.dot(ea_ref[...], w_ref[...], preferred_element_type=jnp.float32)
        + b_ref[...]
    )


def _edge_proj(edge_attr, w_ee, b_e):
    blk = 8000
    return pl.pallas_call(
        _eproj_body,
        grid=(E // blk,),
        in_specs=[
            pl.BlockSpec((blk, DE), lambda i: (i, 0)),
            pl.BlockSpec((DE, HE), lambda i: (0, 0)),
            pl.BlockSpec((1, HE), lambda i: (0, 0)),
        ],
        out_specs=pl.BlockSpec((blk, HE), lambda i: (i, 0)),
        out_shape=jax.ShapeDtypeStruct((E, HE), jnp.float32),
    )(edge_attr, w_ee, b_e)


def _nodeup_body(x_ref, a_ref, wx_ref, wa_ref, b_ref, o_ref):
    acc = jnp.dot(x_ref[...], wx_ref[...], preferred_element_type=jnp.float32)
    agg = a_ref[0] + a_ref[1]
    acc = acc + jnp.dot(agg, wa_ref[...], preferred_element_type=jnp.float32)
    o_ref[...] = jnp.maximum(acc + b_ref[...], 0.0)


def _node_update(x, aggs, w_x, w_a, b_n):
    blk = 1000
    return pl.pallas_call(
        _nodeup_body,
        grid=(N // blk,),
        in_specs=[
            pl.BlockSpec((blk, DF), lambda i: (i, 0)),
            pl.BlockSpec((NC, blk, HE), lambda i: (0, i, 0)),
            pl.BlockSpec((DF, HN), lambda i: (0, 0)),
            pl.BlockSpec((HE, HN), lambda i: (0, 0)),
            pl.BlockSpec((1, HN), lambda i: (0, 0)),
        ],
        out_specs=pl.BlockSpec((blk, HN), lambda i: (i, 0)),
        out_shape=jax.ShapeDtypeStruct((N, HN), jnp.float32),
    )(x, aggs, w_x, w_a, b_n)


def _sc_edge_agg(src, dst, xs, xr, ep, zeros):
    mesh = plsc.VectorSubcoreMesh(core_axis_name="c", subcore_axis_name="s")

    @functools.partial(
        pl.kernel,
        out_type=jax.ShapeDtypeStruct((NC, N, HE), jnp.float32),
        mesh=mesh,
        compiler_params=pltpu.CompilerParams(use_tc_tiling_on_sc=False),
        scratch_types=[
            pltpu.VMEM((EB,), jnp.int32),
            pltpu.VMEM((EB,), jnp.int32),
            pltpu.VMEM((EB, HE), jnp.float32),
            pltpu.VMEM((EB, HE), jnp.float32),
            pltpu.VMEM((EB, HE), jnp.float32),
            pltpu.VMEM((EB, HE), jnp.float32),
            pltpu.VMEM_SHARED((N, HE), jnp.float32),
            pltpu.SemaphoreType.DMA,
            pltpu.SemaphoreType.DMA,
            pltpu.SemaphoreType.DMA,
        ],
    )
    def sc_kernel(src_hbm, dst_hbm, xs_hbm, xr_hbm, ep_hbm, z_hbm, out_hbm,
                  src_v, dst_v, xs_v, xr_v, ep_v, eo_v, agg_sh,
                  sem1, sem2, sem3):
        cid = lax.axis_index("c")
        sid = lax.axis_index("s")
        wid = sid * NC + cid

        @pl.when(sid == 0)
        def _():
            pltpu.sync_copy(z_hbm, agg_sh)

        plsc.subcore_barrier()

        base = wid * EPW

        @pl.loop(0, EPW, step=EB)
        def _(off):
            s = base + off
            cp0 = pltpu.async_copy(src_hbm.at[pl.ds(s, EB)], src_v, sem1)
            cp1 = pltpu.async_copy(dst_hbm.at[pl.ds(s, EB)], dst_v, sem2)
            cp2 = pltpu.async_copy(ep_hbm.at[pl.ds(s, EB)], ep_v, sem3)
            cp0.wait()
            cp1.wait()
            cpg1 = pltpu.async_copy(xs_hbm.at[src_v], xs_v, sem1)
            cpg2 = pltpu.async_copy(xr_hbm.at[dst_v], xr_v, sem2)
            cpg1.wait()
            cpg2.wait()
            cp2.wait()

            @pl.loop(0, EB)
            def _(i):
                v = xs_v[i, :] + xr_v[i, :] + ep_v[i, :]
                eo_v[i, :] = jnp.maximum(v, 0.0)

            pltpu.sync_copy(eo_v, agg_sh.at[dst_v], add=True)

        plsc.subcore_barrier()

        @pl.when(sid < N // NPS)
        def _():
            r0 = sid * NPS
            pltpu.sync_copy(agg_sh.at[pl.ds(r0, NPS)],
                            out_hbm.at[cid, pl.ds(r0, NPS)])

    return sc_kernel(src, dst, xs, xr, ep, zeros)


def kernel(x, edge_attr, edge_index, W_e, b_e, W_n, b_n):
    src = edge_index[0]
    dst = edge_index[1]

    w_ee = W_e[:DE]
    w_sr = W_e[DE:]
    w_s = w_sr[:DF]
    w_r = w_sr[DF:]
    w_cat = jnp.concatenate([w_s, w_r], axis=1)

    xs, xr = _node_proj(x, w_cat)
    ep = _edge_proj(edge_attr, w_ee, b_e.reshape(1, HE))

    zeros = jnp.zeros((N, HE), jnp.float32)
    aggs = _sc_edge_agg(src, dst, xs, xr, ep, zeros)

    w_x = W_n[:DF]
    w_a = W_n[DF:]
    return _node_update(x, aggs, w_x, w_a, b_n.reshape(1, HN))


# speedup vs baseline: 10.4728x; 1.5766x over previous
import functools

import jax
import jax.numpy as jnp
from jax import lax
from jax.experimental import pallas as pl
from jax.experimental.pallas import tpu as pltpu
from jax.experimental.pallas import tpu_sc as plsc

N = 10000
E = 320000
DF = 128
DE = 16
HE = 16
HN = 128

NC = 2
NS = 16
NW = NC * NS
EPW = E // NW
EB = 1000
NPS = 1000


def _proj_body(x_ref, w_ref, xs_ref, xr_ref):
    r = jnp.dot(x_ref[...], w_ref[...], preferred_element_type=jnp.float32)
    xs_ref[...] = r[:, :HE]
    xr_ref[...] = r[:, HE:]


def _node_proj(x, w_sr):
    blk = 1000
    return pl.pallas_call(
        _proj_body,
        grid=(N // blk,),
        in_specs=[
            pl.BlockSpec((blk, DF), lambda i: (i, 0)),
            pl.BlockSpec((DF, 2 * HE), lambda i: (0, 0)),
        ],
        out_specs=[
            pl.BlockSpec((blk, HE), lambda i: (i, 0)),
            pl.BlockSpec((blk, HE), lambda i: (i, 0)),
        ],
        out_shape=[
            jax.ShapeDtypeStruct((N, HE), jnp.float32),
            jax.ShapeDtypeStruct((N, HE), jnp.float32),
        ],
    )(x, w_sr)


def _eproj_body(ea_ref, w_ref, b_ref, o_ref):
    o_ref[...] = (
        jnp.dot(ea_ref[...], w_ref[...], preferred_element_type=jnp.float32)
        + b_ref[...]
    )


def _edge_proj(ea2, w2, b2):
    blk = 1000
    rows = E // 8
    return pl.pallas_call(
        _eproj_body,
        grid=(rows // blk,),
        in_specs=[
            pl.BlockSpec((blk, 8 * DE), lambda i: (i, 0)),
            pl.BlockSpec((8 * DE, 8 * HE), lambda i: (0, 0)),
            pl.BlockSpec((1, 8 * HE), lambda i: (0, 0)),
        ],
        out_specs=pl.BlockSpec((blk, 8 * HE), lambda i: (i, 0)),
        out_shape=jax.ShapeDtypeStruct((rows, 8 * HE), jnp.float32),
    )(ea2, w2, b2)


def _nodeup_body(x_ref, a_ref, wx_ref, wa_ref, b_ref, o_ref):
    acc = jnp.dot(x_ref[...], wx_ref[...], preferred_element_type=jnp.float32)
    agg = a_ref[0] + a_ref[1]
    acc = acc + jnp.dot(agg, wa_ref[...], preferred_element_type=jnp.float32)
    o_ref[...] = jnp.maximum(acc + b_ref[...], 0.0)


def _node_update(x, aggs, w_x, w_a, b_n):
    blk = 1000
    return pl.pallas_call(
        _nodeup_body,
        grid=(N // blk,),
        in_specs=[
            pl.BlockSpec((blk, DF), lambda i: (i, 0)),
            pl.BlockSpec((NC, blk, HE), lambda i: (0, i, 0)),
            pl.BlockSpec((DF, HN), lambda i: (0, 0)),
            pl.BlockSpec((HE, HN), lambda i: (0, 0)),
            pl.BlockSpec((1, HN), lambda i: (0, 0)),
        ],
        out_specs=pl.BlockSpec((blk, HN), lambda i: (i, 0)),
        out_shape=jax.ShapeDtypeStruct((N, HN), jnp.float32),
    )(x, aggs, w_x, w_a, b_n)


def _sc_edge_agg(src, dst, xs, xr, ep, zeros):
    mesh = plsc.VectorSubcoreMesh(core_axis_name="c", subcore_axis_name="s")

    @functools.partial(
        pl.kernel,
        out_type=jax.ShapeDtypeStruct((NC, N, HE), jnp.float32),
        mesh=mesh,
        compiler_params=pltpu.CompilerParams(use_tc_tiling_on_sc=False),
        scratch_types=[
            pltpu.VMEM((EB,), jnp.int32),
            pltpu.VMEM((EB,), jnp.int32),
            pltpu.VMEM((EB, HE), jnp.float32),
            pltpu.VMEM((EB, HE), jnp.float32),
            pltpu.VMEM((EB // 8, 8 * HE), jnp.float32),
            pltpu.VMEM((EB, HE), jnp.float32),
            pltpu.VMEM_SHARED((N, HE), jnp.float32),
            pltpu.SemaphoreType.DMA,
            pltpu.SemaphoreType.DMA,
            pltpu.SemaphoreType.DMA,
        ],
    )
    def sc_kernel(src_hbm, dst_hbm, xs_hbm, xr_hbm, ep_hbm, z_hbm, out_hbm,
                  src_v, dst_v, xs_v, xr_v, ep_v, eo_v, agg_sh,
                  sem1, sem2, sem3):
        cid = lax.axis_index("c")
        sid = lax.axis_index("s")
        wid = sid * NC + cid

        @pl.when(sid == 0)
        def _():
            pltpu.sync_copy(z_hbm, agg_sh)

        plsc.subcore_barrier()

        base = wid * EPW
        rbase = wid * (EPW // 8)

        @pl.loop(0, EPW // EB)
        def _(t):
            s = base + t * EB
            cp0 = pltpu.async_copy(src_hbm.at[pl.ds(s, EB)], src_v, sem1)
            cp1 = pltpu.async_copy(dst_hbm.at[pl.ds(s, EB)], dst_v, sem2)
            cp2 = pltpu.async_copy(
                ep_hbm.at[pl.ds(rbase + t * (EB // 8), EB // 8)], ep_v, sem3)
            cp0.wait()
            cp1.wait()
            cpg1 = pltpu.async_copy(xs_hbm.at[src_v], xs_v, sem1)
            cpg2 = pltpu.async_copy(xr_hbm.at[dst_v], xr_v, sem2)
            cpg1.wait()
            cpg2.wait()
            cp2.wait()

            @pl.loop(0, EB // 8)
            def _(u):
                for j in range(8):
                    i = u * 8 + j
                    v = (xs_v[i, :] + xr_v[i, :]
                         + ep_v[u, pl.ds(j * HE, HE)])
                    eo_v[i, :] = jnp.maximum(v, 0.0)

            pltpu.sync_copy(eo_v, agg_sh.at[dst_v], add=True)

        plsc.subcore_barrier()

        @pl.when(sid < N // NPS)
        def _():
            r0 = sid * NPS
            pltpu.sync_copy(agg_sh.at[pl.ds(r0, NPS)],
                            out_hbm.at[cid, pl.ds(r0, NPS)])

    return sc_kernel(src, dst, xs, xr, ep, zeros)


def kernel(x, edge_attr, edge_index, W_e, b_e, W_n, b_n):
    src = edge_index[0]
    dst = edge_index[1]

    w_ee = W_e[:DE]
    w_sr = W_e[DE:]
    w_s = w_sr[:DF]
    w_r = w_sr[DF:]
    w_cat = jnp.concatenate([w_s, w_r], axis=1)

    xs, xr = _node_proj(x, w_cat)

    ea2 = edge_attr.reshape(E // 8, 8 * DE)
    w2 = jnp.kron(jnp.eye(8, dtype=jnp.float32), w_ee)
    b2 = jnp.tile(b_e, 8).reshape(1, 8 * HE)
    ep = _edge_proj(ea2, w2, b2)

    zeros = jnp.zeros((N, HE), jnp.float32)
    aggs = _sc_edge_agg(src, dst, xs, xr, ep, zeros)

    w_x = W_n[:DF]
    w_a = W_n[DF:]
    return _node_update(x, aggs, w_x, w_a, b_n.reshape(1, HN))
